# baseline (speedup 1.0000x reference)
import functools

import jax
import jax.numpy as jnp
import numpy as np
from jax import lax
from jax.experimental import pallas as pl
from jax.experimental.pallas import tpu as pltpu
from jax.experimental.pallas import tpu_sc as plsc

N = 50000
E = 1600000
EMB = 16
LAT = 8
ED = 16

NC, NS = 2, 16
NW = NC * NS
EPW = E // NW
CH = 2000
NCHUNK = EPW // CH
NPAD = 50048
RPT = NPAD // NS
PK = 8
EP = E // PK
NP8 = NPAD // PK

_mesh = plsc.VectorSubcoreMesh(core_axis_name="c", subcore_axis_name="s")


def _blockdiag(M, p):
    k, n = M.shape
    out = np.zeros((k * p, n * p), np.float32)
    for i in range(p):
        out[i * k:(i + 1) * k, i * n:(i + 1) * n] = M
    return out


_Rnp = np.zeros((EMB, EMB * LAT), np.float32)
_Snp = np.zeros((EMB * LAT, 16), np.float32)
for _d in range(EMB):
    for _o in range(LAT):
        _Rnp[_d, _d * LAT + _o] = 1.0
        _Snp[_d * LAT + _o, _o] = 1.0
_cnp = np.zeros((1, 16), np.float32)
_cnp[0, LAT] = 1.0
_Rb_np = _blockdiag(_Rnp, PK)
_Sb_np = _blockdiag(_Snp, PK)
_ct_np = np.tile(_cnp, (1, PK))

_Bnp = np.zeros((128, 128), np.float32)
for _j in range(PK):
    _Bnp[16 * _j + LAT, 16 * _j:16 * _j + 16] = 1.0
_selcnt_np = np.zeros((1, 128), np.float32)
_sellat_np = np.zeros((1, 128), np.float32)
for _j in range(PK):
    _selcnt_np[0, 16 * _j + LAT] = 1.0
    _sellat_np[0, 16 * _j:16 * _j + LAT] = 1.0


@functools.partial(
    pl.kernel,
    out_type=jax.ShapeDtypeStruct((E, 16), jnp.float32),
    mesh=_mesh,
    compiler_params=pltpu.CompilerParams(use_tc_tiling_on_sc=False),
    scratch_types=[
        pltpu.VMEM((EPW,), jnp.int32),
        pltpu.VMEM((CH, 16), jnp.float32),
        pltpu.VMEM((CH, 16), jnp.float32),
        pltpu.SemaphoreType.DMA,
        pltpu.SemaphoreType.DMA,
    ],
)
def _sc_gather(table_hbm, src_hbm, out_hbm, idx_v, rows0_v, rows1_v, sem0, sem1):
    wid = lax.axis_index("s") * NC + lax.axis_index("c")
    base = wid * EPW
    pltpu.sync_copy(src_hbm.at[pl.ds(base, EPW)], idx_v)
    rows = (rows0_v, rows1_v)
    sems = (sem0, sem1)
    cps = [None, None]
    cps[0] = pltpu.async_copy(table_hbm.at[idx_v.at[pl.ds(0, CH)]], rows[0], sems[0])
    for j in range(NCHUNK):
        cur = j % 2
        nxt = 1 - cur
        if j + 1 < NCHUNK:
            cps[nxt] = pltpu.async_copy(
                table_hbm.at[idx_v.at[pl.ds((j + 1) * CH, CH)]], rows[nxt], sems[nxt])
        cps[cur].wait()
        pltpu.sync_copy(rows[cur], out_hbm.at[pl.ds(base + j * CH, CH)])


CH2 = 1000
NCHUNK2 = EPW // CH2


@functools.partial(
    pl.kernel,
    out_type=[
        jax.ShapeDtypeStruct((E, 16), jnp.float32),
        jax.ShapeDtypeStruct((E, 16), jnp.float32),
    ],
    mesh=_mesh,
    compiler_params=pltpu.CompilerParams(use_tc_tiling_on_sc=False),
    scratch_types=[
        pltpu.VMEM((CH2,), jnp.int32),
        pltpu.VMEM((CH2,), jnp.int32),
        pltpu.VMEM((CH2,), jnp.int32),
        pltpu.VMEM((CH2,), jnp.int32),
        pltpu.VMEM((CH2, 16), jnp.float32),
        pltpu.VMEM((CH2, 16), jnp.float32),
        pltpu.VMEM((CH2, 16), jnp.float32),
        pltpu.VMEM((CH2, 16), jnp.float32),
        pltpu.SemaphoreType.DMA,
        pltpu.SemaphoreType.DMA,
        pltpu.SemaphoreType.DMA,
        pltpu.SemaphoreType.DMA,
    ],
)
def _sc_gather2(table_hbm, src_hbm, dst_hbm, outs_hbm, outd_hbm,
                si0_v, si1_v, di0_v, di1_v, sr0_v, sr1_v, dr0_v, dr1_v,
                sem_s0, sem_s1, sem_d0, sem_d1):
    wid = lax.axis_index("s") * NC + lax.axis_index("c")
    base = wid * EPW
    sidx = (si0_v, si1_v)
    didx = (di0_v, di1_v)
    srows = (sr0_v, sr1_v)
    drows = (dr0_v, dr1_v)
    ssems = (sem_s0, sem_s1)
    dsems = (sem_d0, sem_d1)
    scps = [None, None]
    dcps = [None, None]
    pltpu.sync_copy(src_hbm.at[pl.ds(base, CH2)], sidx[0])
    pltpu.sync_copy(dst_hbm.at[pl.ds(base, CH2)], didx[0])
    scps[0] = pltpu.async_copy(table_hbm.at[sidx[0]], srows[0], ssems[0])
    dcps[0] = pltpu.async_copy(table_hbm.at[didx[0]], drows[0], dsems[0])
    for j in range(NCHUNK2):
        cur = j % 2
        nxt = 1 - cur
        if j + 1 < NCHUNK2:
            pltpu.sync_copy(src_hbm.at[pl.ds(base + (j + 1) * CH2, CH2)], sidx[nxt])
            pltpu.sync_copy(dst_hbm.at[pl.ds(base + (j + 1) * CH2, CH2)], didx[nxt])
            scps[nxt] = pltpu.async_copy(table_hbm.at[sidx[nxt]], srows[nxt], ssems[nxt])
            dcps[nxt] = pltpu.async_copy(table_hbm.at[didx[nxt]], drows[nxt], dsems[nxt])
        scps[cur].wait()
        dcps[cur].wait()
        pltpu.sync_copy(srows[cur], outs_hbm.at[pl.ds(base + j * CH2, CH2)])
        pltpu.sync_copy(drows[cur], outd_hbm.at[pl.ds(base + j * CH2, CH2)])


@functools.partial(
    pl.kernel,
    out_type=jax.ShapeDtypeStruct((NC, NPAD, 16), jnp.float32),
    mesh=_mesh,
    compiler_params=pltpu.CompilerParams(use_tc_tiling_on_sc=False),
    scratch_types=[
        pltpu.VMEM((CH,), jnp.int32),
        pltpu.VMEM((CH,), jnp.int32),
        pltpu.VMEM((CH, 16), jnp.float32),
        pltpu.VMEM((CH, 16), jnp.float32),
        pltpu.VMEM_SHARED((NPAD, 16), jnp.float32),
        pltpu.SemaphoreType.DMA,
        pltpu.SemaphoreType.DMA,
        pltpu.SemaphoreType.DMA,
        pltpu.SemaphoreType.DMA,
    ],
)
def _sc_scatter_add(msg_hbm, dst_hbm, zeros_hbm, out_hbm,
                    idx0_v, idx1_v, rows0_v, rows1_v, acc_sh,
                    semi0, semi1, semr0, semr1):
    cid = lax.axis_index("c")
    sid = lax.axis_index("s")
    wid = sid * NC + cid
    base = wid * EPW
    idx = (idx0_v, idx1_v)
    rows = (rows0_v, rows1_v)
    isems = (semi0, semi1)
    rsems = (semr0, semr1)
    icps = [None, None]
    rcps = [None, None]

    @pl.when(sid == 0)
    def _():
        pltpu.sync_copy(zeros_hbm, acc_sh)

    plsc.subcore_barrier()
    icps[0] = pltpu.async_copy(dst_hbm.at[pl.ds(base, CH)], idx[0], isems[0])
    rcps[0] = pltpu.async_copy(msg_hbm.at[pl.ds(base, CH)], rows[0], rsems[0])
    for j in range(NCHUNK):
        cur = j % 2
        nxt = 1 - cur
        if j + 1 < NCHUNK:
            icps[nxt] = pltpu.async_copy(
                dst_hbm.at[pl.ds(base + (j + 1) * CH, CH)], idx[nxt], isems[nxt])
            rcps[nxt] = pltpu.async_copy(
                msg_hbm.at[pl.ds(base + (j + 1) * CH, CH)], rows[nxt], rsems[nxt])
        icps[cur].wait()
        rcps[cur].wait()
        pltpu.sync_copy(rows[cur], acc_sh.at[idx[cur]], add=True)
    plsc.subcore_barrier()
    pltpu.sync_copy(acc_sh.at[pl.ds(sid * RPT, RPT)],
                    out_hbm.at[cid, pl.ds(sid * RPT, RPT)])


@functools.partial(
    pl.kernel,
    out_type=jax.ShapeDtypeStruct((E, 16), jnp.float32),
    mesh=_mesh,
    compiler_params=pltpu.CompilerParams(use_tc_tiling_on_sc=False),
    scratch_types=[
        pltpu.VMEM((CH, 16), jnp.float32),
        pltpu.VMEM((CH, 16), jnp.float32),
        pltpu.SemaphoreType.DMA,
        pltpu.SemaphoreType.DMA,
    ],
)
def _sc_emit(in_hbm, out_hbm, buf0_v, buf1_v, sem0, sem1):
    wid = lax.axis_index("s") * NC + lax.axis_index("c")
    base = wid * EPW
    bufs = (buf0_v, buf1_v)
    sems = (sem0, sem1)
    cps = [None, None]
    cps[0] = pltpu.async_copy(in_hbm.at[pl.ds(base, CH)], bufs[0], sems[0])
    for j in range(NCHUNK):
        cur = j % 2
        nxt = 1 - cur
        if j + 1 < NCHUNK:
            cps[nxt] = pltpu.async_copy(
                in_hbm.at[pl.ds(base + (j + 1) * CH, CH)], bufs[nxt], sems[nxt])
        cps[cur].wait()
        pltpu.sync_copy(bufs[cur], out_hbm.at[pl.ds(base + j * CH, CH)])


def _tc_msg_body(ea_ref, xs_ref, W1b_ref, b1t_ref, W2b_ref, b2t_ref,
                 Rb_ref, Sb_ref, ct_ref, out_ref):
    a = jnp.maximum(
        jnp.dot(ea_ref[...], W1b_ref[...],
                preferred_element_type=jnp.float32) + b1t_ref[...], 0.0)
    h = jnp.dot(a, W2b_ref[...],
                preferred_element_type=jnp.float32) + b2t_ref[...]
    xe = jnp.dot(xs_ref[...], Rb_ref[...], preferred_element_type=jnp.float32)
    out_ref[...] = jnp.dot(xe * h, Sb_ref[...],
                           preferred_element_type=jnp.float32) + ct_ref[...]


def _tc_latent_body(p_ref, ne_ref, rootb_ref, biast_ref, B_ref,
                    selc_ref, sell_ref, out_ref):
    s = p_ref[0] + p_ref[1]
    bc = jnp.dot(s * selc_ref[...], B_ref[...], preferred_element_type=jnp.float32)
    agg = s * sell_ref[...] / jnp.maximum(bc, 1.0)
    out_ref[...] = agg + jnp.dot(ne_ref[...], rootb_ref[...],
                                 preferred_element_type=jnp.float32) + biast_ref[...]


def _tc_dec_body(ls_ref, ld_ref, P1ab_ref, P1bb_ref, pb1t_ref,
                 P2b_ref, pb2t_ref, out_ref):
    t = (jnp.dot(ls_ref[...], P1ab_ref[...],
                 preferred_element_type=jnp.float32)
         + jnp.dot(ld_ref[...], P1bb_ref[...],
                   preferred_element_type=jnp.float32)
         + pb1t_ref[...])
    out_ref[...] = jnp.dot(jnp.maximum(t, 0.0), P2b_ref[...],
                           preferred_element_type=jnp.float32) + pb2t_ref[...]


_BP = 2000


def _full(shape):
    return pl.BlockSpec(shape, lambda i: (0,) * len(shape))


def kernel(edge_index, edge_attr, node_emb, W1, b1, W2, b2, root, bias,
           P1, pb1, P2, pb2):
    f32 = jnp.float32

    src = edge_index[0]
    dst = edge_index[1]

    x_src = _sc_gather(node_emb, src)

    eye = jnp.asarray(np.eye(PK, dtype=np.float32))
    W1b = jnp.einsum('pq,kn->pkqn', eye, W1).reshape(PK * ED, PK * 64)
    b1t = jnp.tile(b1.reshape(1, 64), (1, PK))
    W2b = jnp.einsum('pq,kn->pkqn', eye, W2).reshape(PK * 64, PK * EMB * LAT)
    b2t = jnp.tile(b2.reshape(1, EMB * LAT), (1, PK))
    Rb = jnp.asarray(_Rb_np)
    Sb = jnp.asarray(_Sb_np)
    ct = jnp.asarray(_ct_np)

    ea_p = edge_attr.reshape(EP, 128)
    xs_p = x_src.reshape(EP, 128)
    grid = (EP // _BP,)
    msg_p = pl.pallas_call(
        _tc_msg_body,
        grid=grid,
        in_specs=[
            pl.BlockSpec((_BP, 128), lambda i: (i, 0)),
            pl.BlockSpec((_BP, 128), lambda i: (i, 0)),
            _full((128, 512)), _full((1, 512)),
            _full((512, 1024)), _full((1, 1024)),
            _full((128, 1024)), _full((1024, 128)), _full((1, 128)),
        ],
        out_specs=pl.BlockSpec((_BP, 128), lambda i: (i, 0)),
        out_shape=jax.ShapeDtypeStruct((EP, 128), f32),
    )(ea_p, xs_p, W1b, b1t, W2b, b2t, Rb, Sb, ct)

    zeros = jnp.zeros((NPAD, 16), f32)
    partials = _sc_scatter_add(msg_p.reshape(E, 16), dst, zeros)

    ne_pad = jnp.zeros((NPAD, EMB), f32).at[:N].set(node_emb)
    root_pad = jnp.zeros((EMB, 16), f32).at[:, :LAT].set(root)
    rootb = jnp.einsum('pq,kn->pkqn', eye, root_pad).reshape(128, 128)
    bias_pad = jnp.zeros((1, 16), f32).at[0, :LAT].set(bias)
    biast = jnp.tile(bias_pad, (1, PK))
    latent_p = pl.pallas_call(
        _tc_latent_body,
        grid=(1,),
        in_specs=[
            _full((NC, NP8, 128)),
            _full((NP8, 128)),
            _full((128, 128)), _full((1, 128)), _full((128, 128)),
            _full((1, 128)), _full((1, 128)),
        ],
        out_specs=_full((NP8, 128)),
        out_shape=jax.ShapeDtypeStruct((NP8, 128), f32),
    )(partials.reshape(NC, NP8, 128), ne_pad.reshape(NP8, 128), rootb, biast,
      jnp.asarray(_Bnp), jnp.asarray(_selcnt_np), jnp.asarray(_sellat_np))

    ls, ld = _sc_gather2(latent_p.reshape(NPAD, 16), src, dst)

    P1a = jnp.zeros((16, 64), f32).at[:LAT].set(P1[:LAT])
    P1b = jnp.zeros((16, 64), f32).at[:LAT].set(P1[LAT:])
    P1ab = jnp.einsum('pq,kn->pkqn', eye, P1a).reshape(128, 512)
    P1bb = jnp.einsum('pq,kn->pkqn', eye, P1b).reshape(128, 512)
    pb1t = jnp.tile(pb1.reshape(1, 64), (1, PK))
    P2b = jnp.einsum('pq,kn->pkqn', eye, P2).reshape(512, 128)
    pb2t = jnp.tile(pb2.reshape(1, ED), (1, PK))
    out_p = pl.pallas_call(
        _tc_dec_body,
        grid=grid,
        in_specs=[
            pl.BlockSpec((_BP, 128), lambda i: (i, 0)),
            pl.BlockSpec((_BP, 128), lambda i: (i, 0)),
            _full((128, 512)), _full((128, 512)), _full((1, 512)),
            _full((512, 128)), _full((1, 128)),
        ],
        out_specs=pl.BlockSpec((_BP, 128), lambda i: (i, 0)),
        out_shape=jax.ShapeDtypeStruct((EP, 128), f32),
    )(ls.reshape(EP, 128), ld.reshape(EP, 128), P1ab, P1bb, pb1t, P2b, pb2t)

    return out_p.reshape(E, ED)


# speedup vs baseline: 1.0193x; 1.0193x over previous
import functools

import jax
import jax.numpy as jnp
import numpy as np
from jax import lax
from jax.experimental import pallas as pl
from jax.experimental.pallas import tpu as pltpu
from jax.experimental.pallas import tpu_sc as plsc

N = 50000
E = 1600000
EMB = 16
LAT = 8
ED = 16

NC, NS = 2, 16
NW = NC * NS
EPW = E // NW
CH = 2000
NCHUNK = EPW // CH
NPAD = 50048
RPT = NPAD // NS
PK = 8
EP = E // PK
NP8 = NPAD // PK

_mesh = plsc.VectorSubcoreMesh(core_axis_name="c", subcore_axis_name="s")


def _blockdiag(M, p):
    k, n = M.shape
    out = np.zeros((k * p, n * p), np.float32)
    for i in range(p):
        out[i * k:(i + 1) * k, i * n:(i + 1) * n] = M
    return out


_Rnp = np.zeros((EMB, EMB * LAT), np.float32)
_Snp = np.zeros((EMB * LAT, 16), np.float32)
for _d in range(EMB):
    for _o in range(LAT):
        _Rnp[_d, _d * LAT + _o] = 1.0
        _Snp[_d * LAT + _o, _o] = 1.0
_cnp = np.zeros((1, 16), np.float32)
_cnp[0, LAT] = 1.0
_Rb_np = _blockdiag(_Rnp, PK)
_Sb_np = _blockdiag(_Snp, PK)
_ct_np = np.tile(_cnp, (1, PK))

_Bnp = np.zeros((128, 128), np.float32)
for _j in range(PK):
    _Bnp[16 * _j + LAT, 16 * _j:16 * _j + 16] = 1.0
_selcnt_np = np.zeros((1, 128), np.float32)
_sellat_np = np.zeros((1, 128), np.float32)
for _j in range(PK):
    _selcnt_np[0, 16 * _j + LAT] = 1.0
    _sellat_np[0, 16 * _j:16 * _j + LAT] = 1.0


@functools.partial(
    pl.kernel,
    out_type=jax.ShapeDtypeStruct((E, 16), jnp.float32),
    mesh=_mesh,
    compiler_params=pltpu.CompilerParams(use_tc_tiling_on_sc=False),
    scratch_types=[
        pltpu.VMEM((EPW,), jnp.int32),
        pltpu.VMEM((CH, 16), jnp.float32),
        pltpu.VMEM((CH, 16), jnp.float32),
        pltpu.SemaphoreType.DMA,
        pltpu.SemaphoreType.DMA,
    ],
)
def _sc_gather(table_hbm, ei_hbm, out_hbm, idx_v, rows0_v, rows1_v, sem0, sem1):
    wid = lax.axis_index("s") * NC + lax.axis_index("c")
    base = wid * EPW
    pltpu.sync_copy(ei_hbm.at[0, pl.ds(base, EPW)], idx_v)
    rows = (rows0_v, rows1_v)
    sems = (sem0, sem1)
    cps = [None, None]
    cps[0] = pltpu.async_copy(table_hbm.at[idx_v.at[pl.ds(0, CH)]], rows[0], sems[0])
    for j in range(NCHUNK):
        cur = j % 2
        nxt = 1 - cur
        if j + 1 < NCHUNK:
            cps[nxt] = pltpu.async_copy(
                table_hbm.at[idx_v.at[pl.ds((j + 1) * CH, CH)]], rows[nxt], sems[nxt])
        cps[cur].wait()
        pltpu.sync_copy(rows[cur], out_hbm.at[pl.ds(base + j * CH, CH)])


CH2 = 1000
NCHUNK2 = EPW // CH2


@functools.partial(
    pl.kernel,
    out_type=[
        jax.ShapeDtypeStruct((E, 16), jnp.float32),
        jax.ShapeDtypeStruct((E, 16), jnp.float32),
    ],
    mesh=_mesh,
    compiler_params=pltpu.CompilerParams(use_tc_tiling_on_sc=False),
    scratch_types=[
        pltpu.VMEM((CH2,), jnp.int32),
        pltpu.VMEM((CH2,), jnp.int32),
        pltpu.VMEM((CH2,), jnp.int32),
        pltpu.VMEM((CH2,), jnp.int32),
        pltpu.VMEM((CH2, 16), jnp.float32),
        pltpu.VMEM((CH2, 16), jnp.float32),
        pltpu.VMEM((CH2, 16), jnp.float32),
        pltpu.VMEM((CH2, 16), jnp.float32),
        pltpu.SemaphoreType.DMA,
        pltpu.SemaphoreType.DMA,
        pltpu.SemaphoreType.DMA,
        pltpu.SemaphoreType.DMA,
    ],
)
def _sc_gather2(table_hbm, ei_hbm, outs_hbm, outd_hbm,
                si0_v, si1_v, di0_v, di1_v, sr0_v, sr1_v, dr0_v, dr1_v,
                sem_s0, sem_s1, sem_d0, sem_d1):
    wid = lax.axis_index("s") * NC + lax.axis_index("c")
    base = wid * EPW
    sidx = (si0_v, si1_v)
    didx = (di0_v, di1_v)
    srows = (sr0_v, sr1_v)
    drows = (dr0_v, dr1_v)
    ssems = (sem_s0, sem_s1)
    dsems = (sem_d0, sem_d1)
    scps = [None, None]
    dcps = [None, None]
    pltpu.sync_copy(ei_hbm.at[0, pl.ds(base, CH2)], sidx[0])
    pltpu.sync_copy(ei_hbm.at[1, pl.ds(base, CH2)], didx[0])
    scps[0] = pltpu.async_copy(table_hbm.at[sidx[0]], srows[0], ssems[0])
    dcps[0] = pltpu.async_copy(table_hbm.at[didx[0]], drows[0], dsems[0])
    for j in range(NCHUNK2):
        cur = j % 2
        nxt = 1 - cur
        if j + 1 < NCHUNK2:
            pltpu.sync_copy(ei_hbm.at[0, pl.ds(base + (j + 1) * CH2, CH2)], sidx[nxt])
            pltpu.sync_copy(ei_hbm.at[1, pl.ds(base + (j + 1) * CH2, CH2)], didx[nxt])
            scps[nxt] = pltpu.async_copy(table_hbm.at[sidx[nxt]], srows[nxt], ssems[nxt])
            dcps[nxt] = pltpu.async_copy(table_hbm.at[didx[nxt]], drows[nxt], dsems[nxt])
        scps[cur].wait()
        dcps[cur].wait()
        pltpu.sync_copy(srows[cur], outs_hbm.at[pl.ds(base + j * CH2, CH2)])
        pltpu.sync_copy(drows[cur], outd_hbm.at[pl.ds(base + j * CH2, CH2)])


@functools.partial(
    pl.kernel,
    out_type=jax.ShapeDtypeStruct((NC, NPAD, 16), jnp.float32),
    mesh=_mesh,
    compiler_params=pltpu.CompilerParams(use_tc_tiling_on_sc=False),
    scratch_types=[
        pltpu.VMEM((CH,), jnp.int32),
        pltpu.VMEM((CH,), jnp.int32),
        pltpu.VMEM((CH, 16), jnp.float32),
        pltpu.VMEM((CH, 16), jnp.float32),
        pltpu.VMEM_SHARED((NPAD, 16), jnp.float32),
        pltpu.SemaphoreType.DMA,
        pltpu.SemaphoreType.DMA,
        pltpu.SemaphoreType.DMA,
        pltpu.SemaphoreType.DMA,
    ],
)
def _sc_scatter_add(msg_hbm, ei_hbm, zeros_hbm, out_hbm,
                    idx0_v, idx1_v, rows0_v, rows1_v, acc_sh,
                    semi0, semi1, semr0, semr1):
    cid = lax.axis_index("c")
    sid = lax.axis_index("s")
    wid = sid * NC + cid
    base = wid * EPW
    idx = (idx0_v, idx1_v)
    rows = (rows0_v, rows1_v)
    isems = (semi0, semi1)
    rsems = (semr0, semr1)
    icps = [None, None]
    rcps = [None, None]

    @pl.when(sid == 0)
    def _():
        pltpu.sync_copy(zeros_hbm, acc_sh)

    plsc.subcore_barrier()
    icps[0] = pltpu.async_copy(ei_hbm.at[1, pl.ds(base, CH)], idx[0], isems[0])
    rcps[0] = pltpu.async_copy(msg_hbm.at[pl.ds(base, CH)], rows[0], rsems[0])
    for j in range(NCHUNK):
        cur = j % 2
        nxt = 1 - cur
        if j + 1 < NCHUNK:
            icps[nxt] = pltpu.async_copy(
                ei_hbm.at[1, pl.ds(base + (j + 1) * CH, CH)], idx[nxt], isems[nxt])
            rcps[nxt] = pltpu.async_copy(
                msg_hbm.at[pl.ds(base + (j + 1) * CH, CH)], rows[nxt], rsems[nxt])
        icps[cur].wait()
        rcps[cur].wait()
        pltpu.sync_copy(rows[cur], acc_sh.at[idx[cur]], add=True)
    plsc.subcore_barrier()
    pltpu.sync_copy(acc_sh.at[pl.ds(sid * RPT, RPT)],
                    out_hbm.at[cid, pl.ds(sid * RPT, RPT)])


@functools.partial(
    pl.kernel,
    out_type=jax.ShapeDtypeStruct((E, 16), jnp.float32),
    mesh=_mesh,
    compiler_params=pltpu.CompilerParams(use_tc_tiling_on_sc=False),
    scratch_types=[
        pltpu.VMEM((CH, 16), jnp.float32),
        pltpu.VMEM((CH, 16), jnp.float32),
        pltpu.SemaphoreType.DMA,
        pltpu.SemaphoreType.DMA,
    ],
)
def _sc_emit(in_hbm, out_hbm, buf0_v, buf1_v, sem0, sem1):
    wid = lax.axis_index("s") * NC + lax.axis_index("c")
    base = wid * EPW
    bufs = (buf0_v, buf1_v)
    sems = (sem0, sem1)
    cps = [None, None]
    cps[0] = pltpu.async_copy(in_hbm.at[pl.ds(base, CH)], bufs[0], sems[0])
    for j in range(NCHUNK):
        cur = j % 2
        nxt = 1 - cur
        if j + 1 < NCHUNK:
            cps[nxt] = pltpu.async_copy(
                in_hbm.at[pl.ds(base + (j + 1) * CH, CH)], bufs[nxt], sems[nxt])
        cps[cur].wait()
        pltpu.sync_copy(bufs[cur], out_hbm.at[pl.ds(base + j * CH, CH)])


def _tc_msg_body(ea_ref, xs_ref, W1b_ref, b1t_ref, W2b_ref, b2t_ref,
                 Rb_ref, Sb_ref, ct_ref, out_ref):
    a = jnp.maximum(
        jnp.dot(ea_ref[...], W1b_ref[...],
                preferred_element_type=jnp.float32) + b1t_ref[...], 0.0)
    h = jnp.dot(a, W2b_ref[...],
                preferred_element_type=jnp.float32) + b2t_ref[...]
    xe = jnp.dot(xs_ref[...], Rb_ref[...], preferred_element_type=jnp.float32)
    out_ref[...] = jnp.dot(xe * h, Sb_ref[...],
                           preferred_element_type=jnp.float32) + ct_ref[...]


def _tc_latent_body(p_ref, ne_ref, rootb_ref, biast_ref, B_ref,
                    selc_ref, sell_ref, out_ref):
    s = p_ref[0] + p_ref[1]
    bc = jnp.dot(s * selc_ref[...], B_ref[...], preferred_element_type=jnp.float32)
    agg = s * sell_ref[...] / jnp.maximum(bc, 1.0)
    out_ref[...] = agg + jnp.dot(ne_ref[...], rootb_ref[...],
                                 preferred_element_type=jnp.float32) + biast_ref[...]


def _tc_dec_body(ls_ref, ld_ref, P1ab_ref, P1bb_ref, pb1t_ref,
                 P2b_ref, pb2t_ref, out_ref):
    t = (jnp.dot(ls_ref[...], P1ab_ref[...],
                 preferred_element_type=jnp.float32)
         + jnp.dot(ld_ref[...], P1bb_ref[...],
                   preferred_element_type=jnp.float32)
         + pb1t_ref[...])
    out_ref[...] = jnp.dot(jnp.maximum(t, 0.0), P2b_ref[...],
                           preferred_element_type=jnp.float32) + pb2t_ref[...]


_BP = 4000


def _full(shape):
    return pl.BlockSpec(shape, lambda i: (0,) * len(shape))


def kernel(edge_index, edge_attr, node_emb, W1, b1, W2, b2, root, bias,
           P1, pb1, P2, pb2):
    f32 = jnp.float32

    x_src = _sc_gather(node_emb, edge_index)

    eye = jnp.asarray(np.eye(PK, dtype=np.float32))
    W1b = jnp.einsum('pq,kn->pkqn', eye, W1).reshape(PK * ED, PK * 64)
    b1t = jnp.tile(b1.reshape(1, 64), (1, PK))
    W2b = jnp.einsum('pq,kn->pkqn', eye, W2).reshape(PK * 64, PK * EMB * LAT)
    b2t = jnp.tile(b2.reshape(1, EMB * LAT), (1, PK))
    Rb = jnp.asarray(_Rb_np)
    Sb = jnp.asarray(_Sb_np)
    ct = jnp.asarray(_ct_np)

    ea_p = edge_attr.reshape(EP, 128)
    xs_p = x_src.reshape(EP, 128)
    grid = (EP // _BP,)
    msg_p = pl.pallas_call(
        _tc_msg_body,
        grid=grid,
        in_specs=[
            pl.BlockSpec((_BP, 128), lambda i: (i, 0)),
            pl.BlockSpec((_BP, 128), lambda i: (i, 0)),
            _full((128, 512)), _full((1, 512)),
            _full((512, 1024)), _full((1, 1024)),
            _full((128, 1024)), _full((1024, 128)), _full((1, 128)),
        ],
        out_specs=pl.BlockSpec((_BP, 128), lambda i: (i, 0)),
        out_shape=jax.ShapeDtypeStruct((EP, 128), f32),
    )(ea_p, xs_p, W1b, b1t, W2b, b2t, Rb, Sb, ct)

    zeros = jnp.zeros((NPAD, 16), f32)
    partials = _sc_scatter_add(msg_p.reshape(E, 16), edge_index, zeros)

    ne_pad = jnp.zeros((NPAD, EMB), f32).at[:N].set(node_emb)
    root_pad = jnp.zeros((EMB, 16), f32).at[:, :LAT].set(root)
    rootb = jnp.einsum('pq,kn->pkqn', eye, root_pad).reshape(128, 128)
    bias_pad = jnp.zeros((1, 16), f32).at[0, :LAT].set(bias)
    biast = jnp.tile(bias_pad, (1, PK))
    latent_p = pl.pallas_call(
        _tc_latent_body,
        grid=(1,),
        in_specs=[
            _full((NC, NP8, 128)),
            _full((NP8, 128)),
            _full((128, 128)), _full((1, 128)), _full((128, 128)),
            _full((1, 128)), _full((1, 128)),
        ],
        out_specs=_full((NP8, 128)),
        out_shape=jax.ShapeDtypeStruct((NP8, 128), f32),
    )(partials.reshape(NC, NP8, 128), ne_pad.reshape(NP8, 128), rootb, biast,
      jnp.asarray(_Bnp), jnp.asarray(_selcnt_np), jnp.asarray(_sellat_np))

    ls, ld = _sc_gather2(latent_p.reshape(NPAD, 16), edge_index)

    P1a = jnp.zeros((16, 64), f32).at[:LAT].set(P1[:LAT])
    P1b = jnp.zeros((16, 64), f32).at[:LAT].set(P1[LAT:])
    P1ab = jnp.einsum('pq,kn->pkqn', eye, P1a).reshape(128, 512)
    P1bb = jnp.einsum('pq,kn->pkqn', eye, P1b).reshape(128, 512)
    pb1t = jnp.tile(pb1.reshape(1, 64), (1, PK))
    P2b = jnp.einsum('pq,kn->pkqn', eye, P2).reshape(512, 128)
    pb2t = jnp.tile(pb2.reshape(1, ED), (1, PK))
    out_p = pl.pallas_call(
        _tc_dec_body,
        grid=grid,
        in_specs=[
            pl.BlockSpec((_BP, 128), lambda i: (i, 0)),
            pl.BlockSpec((_BP, 128), lambda i: (i, 0)),
            _full((128, 512)), _full((128, 512)), _full((1, 512)),
            _full((512, 128)), _full((1, 128)),
        ],
        out_specs=pl.BlockSpec((_BP, 128), lambda i: (i, 0)),
        out_shape=jax.ShapeDtypeStruct((EP, 128), f32),
    )(ls.reshape(EP, 128), ld.reshape(EP, 128), P1ab, P1bb, pb1t, P2b, pb2t)

    return out_p.reshape(E, ED)
